# Initial kernel scaffold; baseline (speedup 1.0000x reference)
import jax
import jax.numpy as jnp
from jax.experimental import pallas as pl


def kernel(x, pos, c1_w1, c1_b1, c1_g1, c1_be1, c1_w2, c1_b2, c1_g2, c1_be2, c2_w1, c2_b1, c2_g1, c2_be1, c2_w2, c2_b2, c2_g2, c2_be2, c3_w1, c3_b1, c3_g1, c3_be1, c3_w2, c3_b2, c3_g2, c3_be2, f_w1, f_b1, f_g1, f_be1, f_w2, f_b2, f_g2, f_be2, f_w3, f_b3):
    raise NotImplementedError("write your pallas kernel here")


# speedup vs baseline: 4.9688x; 4.9688x over previous
import functools

import jax
import jax.numpy as jnp
from jax import lax
from jax.experimental import pallas as pl
from jax.experimental.pallas import tpu as pltpu
from jax.experimental.pallas import tpu_sc as plsc

KNN = 32
EPS = 1e-5


def _knn_body(pos_ref, out_ref):
    b = pl.program_id(0)
    p = pos_ref[0]
    n = p.shape[0]
    g = lax.dot_general(p, p, (((1,), (1,)), ((), ())),
                        preferred_element_type=jnp.float32)
    pp = p * p
    sq_r = jnp.sum(pp, axis=1, keepdims=True)
    negd = 2.0 * g - sq_r - sq_r.T
    iota = lax.broadcasted_iota(jnp.int32, (n, n), 1)
    col32 = lax.broadcasted_iota(jnp.int32, (n, KNN), 1)
    base = b * n

    def step(t, carry):
        nd, acc = carry
        m = jnp.max(nd, axis=1, keepdims=True)
        cand = jnp.where(nd == m, iota, n)
        am = jnp.min(cand, axis=1, keepdims=True)
        acc = jnp.where(col32 == t, am + base, acc)
        nd = jnp.where(iota == am, -jnp.inf, nd)
        return nd, acc

    _, acc = lax.fori_loop(0, KNN, step, (negd, jnp.zeros((n, KNN), jnp.int32)))
    out_ref[0] = acc


def _knn(pos_p):
    B, N, _ = pos_p.shape
    return pl.pallas_call(
        _knn_body,
        grid=(B,),
        in_specs=[pl.BlockSpec((1, N, 8), lambda b: (b, 0, 0))],
        out_specs=pl.BlockSpec((1, N, KNN), lambda b: (b, 0, 0)),
        out_shape=jax.ShapeDtypeStruct((B, N, KNN), jnp.int32),
    )(pos_p)


def _gather_rows(table, gidx):
    R = gidx.shape[0]
    h = table.shape[1]
    NW = 32
    per_w = R // NW
    CH = 128
    n_ch = per_w // CH
    mesh = plsc.VectorSubcoreMesh(core_axis_name="c", subcore_axis_name="s")

    @functools.partial(
        pl.kernel, mesh=mesh,
        out_type=jax.ShapeDtypeStruct((R, h), jnp.float32),
        scratch_types=[
            pltpu.VMEM((CH,), jnp.int32),
            pltpu.VMEM((CH, h), jnp.float32),
            pltpu.SemaphoreType.DMA,
        ],
    )
    def k(table_hbm, idx_hbm, out_hbm, idx_v, rows_v, sem):
        wid = lax.axis_index("s") * 2 + lax.axis_index("c")
        base = wid * per_w

        def body(i, c):
            off = base + i * CH
            pltpu.sync_copy(idx_hbm.at[pl.ds(off, CH)], idx_v)
            pltpu.async_copy(table_hbm.at[idx_v], rows_v, sem).wait()
            pltpu.sync_copy(rows_v, out_hbm.at[pl.ds(off, CH)])
            return c

        lax.fori_loop(0, n_ch, body, 0)

    return k(table, gidx)


def _expand_rows(a, rt):
    t, h = a.shape
    return jnp.broadcast_to(a[:, None, :], (t, KNN, h)).reshape(rt, h)


def _edge_y1(xj_ref, f_ref, w1_ref, b1_ref, c_eff):
    rt = xj_ref.shape[0]
    xj = xj_ref[...]
    xi = _expand_rows(f_ref[...], rt)
    if c_eff < 128:
        xj = xj[:, :c_eff]
        xi = xi[:, :c_eff]
    e = jnp.concatenate([xj - xi, xi], axis=1)
    y = jnp.dot(e, w1_ref[...], preferred_element_type=jnp.float32)
    return y + b1_ref[...]


def _bn_affine(y, s_ref, g_ref, be_ref, count):
    mean = s_ref[0:1] * (1.0 / count)
    var = s_ref[1:2] * (1.0 / count) - mean * mean
    sd = jnp.sqrt(var + EPS)
    return (y - mean) / sd * g_ref[...] + be_ref[...]


def _lrelu(f):
    return jnp.where(f >= 0, f, 0.2 * f)


def _stats_body(c_eff, xj_ref, f_ref, w1_ref, b1_ref, out_ref, acc_ref):
    g = pl.program_id(0)
    y = _edge_y1(xj_ref, f_ref, w1_ref, b1_ref, c_eff)
    s = jnp.sum(y, axis=0, keepdims=True)
    q = jnp.sum(y * y, axis=0, keepdims=True)

    @pl.when(g == 0)
    def _():
        acc_ref[...] = jnp.zeros_like(acc_ref)

    acc_ref[0:1] += s
    acc_ref[1:2] += q

    @pl.when(g == pl.num_programs(0) - 1)
    def _():
        out_ref[...] = acc_ref[...]


def _stats(xj, f128, w1, b1, c_eff, t_pts):
    n_pts = f128.shape[0]
    h = w1.shape[1]
    rt = t_pts * KNN
    grid = n_pts // t_pts
    return pl.pallas_call(
        functools.partial(_stats_body, c_eff),
        grid=(grid,),
        in_specs=[
            pl.BlockSpec((rt, 128), lambda i: (i, 0)),
            pl.BlockSpec((t_pts, 128), lambda i: (i, 0)),
            pl.BlockSpec(w1.shape, lambda i: (0, 0)),
            pl.BlockSpec((1, h), lambda i: (0, 0)),
        ],
        out_specs=pl.BlockSpec((8, h), lambda i: (0, 0)),
        out_shape=jax.ShapeDtypeStruct((8, h), jnp.float32),
        scratch_shapes=[pltpu.VMEM((8, h), jnp.float32)],
    )(xj, f128, w1, b1)


def _main_body(count1, c_eff, xj_ref, f_ref, w1_ref, b1_ref, s1_ref, g1_ref,
               be1_ref, w2_ref, b2_ref, mx_ref, mn_ref, out2_ref, acc_ref):
    g = pl.program_id(0)
    t = f_ref.shape[0]
    y1 = _edge_y1(xj_ref, f_ref, w1_ref, b1_ref, c_eff)
    h1 = _lrelu(_bn_affine(y1, s1_ref, g1_ref, be1_ref, count1))
    y2 = jnp.dot(h1, w2_ref[...], preferred_element_type=jnp.float32)
    y2 = y2 + b2_ref[...]
    s = jnp.sum(y2, axis=0, keepdims=True)
    q = jnp.sum(y2 * y2, axis=0, keepdims=True)

    @pl.when(g == 0)
    def _():
        acc_ref[...] = jnp.zeros_like(acc_ref)

    acc_ref[0:1] += s
    acc_ref[1:2] += q

    y3 = y2.reshape(t, KNN, y2.shape[1])
    mx_ref[...] = jnp.max(y3, axis=1)
    mn_ref[...] = jnp.min(y3, axis=1)

    @pl.when(g == pl.num_programs(0) - 1)
    def _():
        out2_ref[...] = acc_ref[...]


def _main(xj, f128, w1, b1, s1, g1, be1, w2, b2, c_eff, t_pts):
    R = xj.shape[0]
    n_pts = f128.shape[0]
    h = w1.shape[1]
    co = w2.shape[1]
    rt = t_pts * KNN
    grid = n_pts // t_pts
    return pl.pallas_call(
        functools.partial(_main_body, float(R), c_eff),
        grid=(grid,),
        in_specs=[
            pl.BlockSpec((rt, 128), lambda i: (i, 0)),
            pl.BlockSpec((t_pts, 128), lambda i: (i, 0)),
            pl.BlockSpec(w1.shape, lambda i: (0, 0)),
            pl.BlockSpec((1, h), lambda i: (0, 0)),
            pl.BlockSpec((8, h), lambda i: (0, 0)),
            pl.BlockSpec((1, h), lambda i: (0, 0)),
            pl.BlockSpec((1, h), lambda i: (0, 0)),
            pl.BlockSpec((h, co), lambda i: (0, 0)),
            pl.BlockSpec((1, co), lambda i: (0, 0)),
        ],
        out_specs=[
            pl.BlockSpec((t_pts, co), lambda i: (i, 0)),
            pl.BlockSpec((t_pts, co), lambda i: (i, 0)),
            pl.BlockSpec((8, co), lambda i: (0, 0)),
        ],
        out_shape=[
            jax.ShapeDtypeStruct((n_pts, co), jnp.float32),
            jax.ShapeDtypeStruct((n_pts, co), jnp.float32),
            jax.ShapeDtypeStruct((8, co), jnp.float32),
        ],
        scratch_shapes=[pltpu.VMEM((8, co), jnp.float32)],
    )(xj, f128, w1, b1, s1, g1, be1, w2, b2)


def _pool_bn(mx_ref, mn_ref, s2_ref, g2_ref, be2_ref, count):
    sel = jnp.where(g2_ref[...] >= 0, mx_ref[...], mn_ref[...])
    return _lrelu(_bn_affine(sel, s2_ref, g2_ref, be2_ref, count))


def _fin_body(count, mx_ref, mn_ref, s2_ref, g2_ref, be2_ref, out_ref):
    f = _pool_bn(mx_ref, mn_ref, s2_ref, g2_ref, be2_ref, count)
    co = f.shape[1]
    if co < 128:
        f = jnp.concatenate([f, jnp.zeros((f.shape[0], 128 - co), f.dtype)],
                            axis=1)
    out_ref[...] = f


def _fin(mx, mn, s2, g2, be2, count):
    n_pts = mx.shape[0]
    return pl.pallas_call(
        functools.partial(_fin_body, count),
        out_shape=jax.ShapeDtypeStruct((n_pts, 128), jnp.float32),
    )(mx, mn, s2, g2, be2)


def _bn_lrelu(y, gg, bb):
    n = y.shape[0]
    mean = jnp.sum(y, axis=0, keepdims=True) * (1.0 / n)
    var = jnp.sum(y * y, axis=0, keepdims=True) * (1.0 / n) - mean * mean
    sd = jnp.sqrt(var + EPS)
    return _lrelu((y - mean) / sd * gg + bb)


def _head_body(count, mx_ref, mn_ref, s2_ref, g2_ref, be2_ref,
               w1_ref, b1_ref, g1_ref, be1_ref,
               w2_ref, b2_ref, gg2_ref, bee2_ref,
               w3_ref, b3_ref, out_ref):
    f = _pool_bn(mx_ref, mn_ref, s2_ref, g2_ref, be2_ref, count)
    y = jnp.dot(f, w1_ref[...], preferred_element_type=jnp.float32) + b1_ref[...]
    f = _bn_lrelu(y, g1_ref[...], be1_ref[...])
    y = jnp.dot(f, w2_ref[...], preferred_element_type=jnp.float32) + b2_ref[...]
    f = _bn_lrelu(y, gg2_ref[...], bee2_ref[...])
    out_ref[...] = (jnp.dot(f, w3_ref[...], preferred_element_type=jnp.float32)
                    + b3_ref[...])


def _head(mx, mn, s2, g2, be2, w1, b1, g1, be1, w2, b2, gg2, bee2, w3, b3,
          count):
    n_pts = mx.shape[0]
    return pl.pallas_call(
        functools.partial(_head_body, count),
        out_shape=jax.ShapeDtypeStruct((n_pts, w3.shape[1]), jnp.float32),
    )(mx, mn, s2, g2, be2, w1, b1, g1, be1, w2, b2, gg2, bee2, w3, b3)


def _edge_block(f128, gidx, w1, b1, g1, be1, w2, b2, c_eff, t_pts):
    R = float(gidx.shape[0])
    xj = _gather_rows(f128, gidx)
    s1 = _stats(xj, f128, w1, b1[None], c_eff, t_pts)
    return _main(xj, f128, w1, b1[None], s1, g1[None], be1[None], w2, b2[None],
                 c_eff, t_pts)


def kernel(x, pos, c1_w1, c1_b1, c1_g1, c1_be1, c1_w2, c1_b2, c1_g2, c1_be2,
           c2_w1, c2_b1, c2_g1, c2_be1, c2_w2, c2_b2, c2_g2, c2_be2,
           c3_w1, c3_b1, c3_g1, c3_be1, c3_w2, c3_b2, c3_g2, c3_be2,
           f_w1, f_b1, f_g1, f_be1, f_w2, f_b2, f_g2, f_be2, f_w3, f_b3):
    B, N, _ = x.shape
    n_pts = B * N
    R = float(n_pts * KNN)

    pos_p = jnp.pad(pos, ((0, 0), (0, 0), (0, 5)))
    gidx = _knn(pos_p).reshape(n_pts * KNN)

    f1 = jnp.pad(x.reshape(n_pts, 3), ((0, 0), (0, 125)))
    w1p = jnp.concatenate([
        jnp.pad(c1_w1[:3], ((0, 125), (0, 0))),
        jnp.pad(c1_w1[3:], ((0, 125), (0, 0))),
    ])
    mx, mn, s2 = _edge_block(f1, gidx, w1p, c1_b1, c1_g1, c1_be1, c1_w2,
                             c1_b2, 128, 64)
    f2 = _fin(mx, mn, s2, c1_g2[None], c1_be2[None], R)

    mx, mn, s2 = _edge_block(f2, gidx, c2_w1, c2_b1, c2_g1, c2_be1, c2_w2,
                             c2_b2, 64, 64)
    f3 = _fin(mx, mn, s2, c2_g2[None], c2_be2[None], R)

    mx, mn, s2 = _edge_block(f3, gidx, c3_w1, c3_b1, c3_g1, c3_be1, c3_w2,
                             c3_b2, 128, 16)

    out = _head(mx, mn, s2, c3_g2[None], c3_be2[None],
                f_w1, f_b1[None], f_g1[None], f_be1[None],
                f_w2, f_b2[None], f_g2[None], f_be2[None],
                f_w3, f_b3[None], R)
    return out.reshape(B, N, 12)
